# baseline (speedup 1.0000x reference)
import functools

import jax
import jax.numpy as jnp
from jax import lax
from jax.experimental import pallas as pl
from jax.experimental.pallas import tpu as pltpu
from jax.experimental.pallas import tpu_sc as plsc

B, S, N = 2, 2048, 1024
TD, SD, H, K = 1024, 1024, 16, 8
HD = SD // H
SCALE = HD ** (-0.5)
BS = 256
M = B * S * K
KV = 2 * SD
NC, NS = 2, 16
NW = NC * NS
PER_W = M // NW
CH = 64


def _route_body(tok_ref, statesT_ref, rwT_ref, rb_ref, qwT_ref, qb_ref,
                idx_ref, q_ref):
    b = pl.program_id(0)
    tok = tok_ref[0]
    routed = jnp.dot(tok, rwT_ref[...], preferred_element_type=jnp.float32) + rb_ref[...]
    scores = jnp.dot(routed, statesT_ref[0], preferred_element_type=jnp.float32)

    iota_n = lax.broadcasted_iota(jnp.int32, (BS, N), 1)
    x = scores
    idxs = []
    for _ in range(K):
        m = jnp.max(x, axis=-1, keepdims=True)
        am = jnp.min(jnp.where(x == m, iota_n, N), axis=-1, keepdims=True)
        idxs.append(am)
        x = jnp.where(iota_n == am, -3e38, x)

    idx_ref[0] = jnp.concatenate(idxs, axis=1) + b * N
    q_ref[0] = jnp.dot(tok.astype(jnp.bfloat16), qwT_ref[...].astype(jnp.bfloat16),
                       preferred_element_type=jnp.float32) + qb_ref[...]


def _kv_body(states_ref, kwT_ref, vwT_ref, kb_ref, vb_ref, kv_ref):
    st = states_ref[0]
    kst = jnp.dot(st, kwT_ref[...], preferred_element_type=jnp.float32) + kb_ref[...]
    vst = jnp.dot(st, vwT_ref[...], preferred_element_type=jnp.float32) + vb_ref[...]
    kv_ref[0, :, :SD] = kst.astype(jnp.bfloat16)
    kv_ref[0, :, SD:] = vst.astype(jnp.bfloat16)


def _sc_gather(table, idx):
    mesh = plsc.VectorSubcoreMesh(core_axis_name="c", subcore_axis_name="s")

    @functools.partial(
        pl.kernel,
        mesh=mesh,
        out_type=jax.ShapeDtypeStruct((M, SD), jnp.int32),
        scratch_types=[
            pltpu.VMEM((CH,), jnp.int32),
            pltpu.VMEM((CH, SD), jnp.int32),
            pltpu.SemaphoreType.DMA,
        ],
    )
    def gather_k(table_hbm, idx_hbm, out_hbm, idx_v, rows_v, sem):
        wid = lax.axis_index("s") * NC + lax.axis_index("c")
        base = wid * PER_W

        def body(i, carry):
            off = base + i * CH
            pltpu.sync_copy(idx_hbm.at[pl.ds(off, CH)], idx_v)
            pltpu.async_copy(table_hbm.at[idx_v], rows_v, sem).wait()
            pltpu.sync_copy(rows_v, out_hbm.at[pl.ds(off, CH)])
            return carry

        lax.fori_loop(0, PER_W // CH, body, 0)

    return gather_k(table, idx)


def _attn_body(q_ref, kv_ref, owT_ref, ob_ref, out_ref, attn_ref):
    q = q_ref[0]

    d_iota = lax.broadcasted_iota(jnp.int32, (SD, H), 0)
    h_iota = lax.broadcasted_iota(jnp.int32, (SD, H), 1)
    dmat = (d_iota // HD == h_iota).astype(jnp.float32)
    h2 = lax.broadcasted_iota(jnp.int32, (H, SD), 0)
    d2 = lax.broadcasted_iota(jnp.int32, (H, SD), 1)
    dmat2 = (d2 // HD == h2).astype(jnp.float32)

    logits = []
    for kk in range(K):
        ksel = kv_ref[0, :, kk * KV:kk * KV + SD].astype(jnp.float32)
        logits.append(jnp.dot(ksel * q, dmat, preferred_element_type=jnp.float32) * SCALE)

    mx = logits[0]
    for kk in range(1, K):
        mx = jnp.maximum(mx, logits[kk])
    es = [jnp.exp(l - mx) for l in logits]
    denom = es[0]
    for kk in range(1, K):
        denom = denom + es[kk]

    o = jnp.zeros((BS, SD), jnp.float32)
    for kk in range(K):
        w = es[kk] / denom
        attn_ref[0, kk] = w
        vsel = kv_ref[0, :, kk * KV + SD:(kk + 1) * KV].astype(jnp.float32)
        wexp = jnp.dot(w, dmat2, preferred_element_type=jnp.float32)
        o = o + wexp * vsel

    out_ref[0] = jnp.dot(o.astype(jnp.bfloat16), owT_ref[...].astype(jnp.bfloat16),
                         preferred_element_type=jnp.float32) + ob_ref[...]


def kernel(tokens, states, router_w, router_b, q_w, q_b, k_w, k_b, v_w, v_b, out_w, out_b):
    statesT = states.transpose(0, 2, 1)
    rwT, qwT, kwT, vwT, owT = router_w.T, q_w.T, k_w.T, v_w.T, out_w.T
    rb = router_b.reshape(1, SD)
    qb = q_b.reshape(1, SD)
    kb = k_b.reshape(1, SD)
    vb = v_b.reshape(1, SD)
    ob = out_b.reshape(1, SD)

    idx, q = pl.pallas_call(
        _route_body,
        grid=(B, S // BS),
        in_specs=[
            pl.BlockSpec((1, BS, TD), lambda b, s: (b, s, 0)),
            pl.BlockSpec((1, SD, N), lambda b, s: (b, 0, 0)),
            pl.BlockSpec((TD, SD), lambda b, s: (0, 0)),
            pl.BlockSpec((1, SD), lambda b, s: (0, 0)),
            pl.BlockSpec((TD, SD), lambda b, s: (0, 0)),
            pl.BlockSpec((1, SD), lambda b, s: (0, 0)),
        ],
        out_specs=[
            pl.BlockSpec((1, BS, K), lambda b, s: (b, s, 0)),
            pl.BlockSpec((1, BS, SD), lambda b, s: (b, s, 0)),
        ],
        out_shape=[
            jax.ShapeDtypeStruct((B, S, K), jnp.int32),
            jax.ShapeDtypeStruct((B, S, SD), jnp.float32),
        ],
    )(tokens, statesT, rwT, rb, qwT, qb)

    kv = pl.pallas_call(
        _kv_body,
        grid=(B,),
        in_specs=[
            pl.BlockSpec((1, N, SD), lambda b: (b, 0, 0)),
            pl.BlockSpec((SD, SD), lambda b: (0, 0)),
            pl.BlockSpec((SD, SD), lambda b: (0, 0)),
            pl.BlockSpec((1, SD), lambda b: (0, 0)),
            pl.BlockSpec((1, SD), lambda b: (0, 0)),
        ],
        out_specs=pl.BlockSpec((1, N, KV), lambda b: (b, 0, 0)),
        out_shape=jax.ShapeDtypeStruct((B, N, KV), jnp.bfloat16),
    )(states, kwT, vwT, kb, vb)

    table_i32 = lax.bitcast_convert_type(kv.reshape(B * N, SD, 2), jnp.int32)
    gathered = _sc_gather(table_i32, idx.reshape(M))
    kv_sel = lax.bitcast_convert_type(gathered, jnp.bfloat16).reshape(B, S, K * KV)

    out, attn = pl.pallas_call(
        _attn_body,
        grid=(B, S // BS),
        in_specs=[
            pl.BlockSpec((1, BS, SD), lambda b, s: (b, s, 0)),
            pl.BlockSpec((1, BS, K * KV), lambda b, s: (b, s, 0)),
            pl.BlockSpec((SD, SD), lambda b, s: (0, 0)),
            pl.BlockSpec((1, SD), lambda b, s: (0, 0)),
        ],
        out_specs=[
            pl.BlockSpec((1, BS, SD), lambda b, s: (b, s, 0)),
            pl.BlockSpec((1, K, BS, H), lambda b, s: (b, 0, s, 0)),
        ],
        out_shape=[
            jax.ShapeDtypeStruct((B, S, SD), jnp.float32),
            jax.ShapeDtypeStruct((B, K, S, H), jnp.float32),
        ],
    )(q, kv_sel, owT, ob)

    return (out, attn.transpose(0, 3, 2, 1))


# speedup vs baseline: 80.0334x; 80.0334x over previous
import functools

import jax
import jax.numpy as jnp
from jax import lax
from jax.experimental import pallas as pl
from jax.experimental.pallas import tpu as pltpu
from jax.experimental.pallas import tpu_sc as plsc

B, S, N = 2, 2048, 1024
TD, SD, H, K = 1024, 1024, 16, 8
HD = SD // H
SCALE = HD ** (-0.5)
BS = 256
BSK = BS * K
M = B * S * K
KV = 2 * SD
NC, NS = 2, 16
NW = NC * NS
PER_W = M // NW
CH = 32


def _route_body(tok_ref, statesT_ref, rwT_ref, rb_ref, qwT_ref, qb_ref,
                idx_ref, q_ref):
    b = pl.program_id(0)
    tok = tok_ref[0]
    routed = jnp.dot(tok, rwT_ref[...], preferred_element_type=jnp.float32) + rb_ref[...]
    scores = jnp.dot(routed, statesT_ref[0], preferred_element_type=jnp.float32)

    iota_n = lax.broadcasted_iota(jnp.int32, (BS, N), 1)
    x = scores
    idxs = []
    for _ in range(K):
        m = jnp.max(x, axis=-1, keepdims=True)
        am = jnp.min(jnp.where(x == m, iota_n, N), axis=-1, keepdims=True)
        idxs.append(am)
        x = jnp.where(iota_n == am, -3e38, x)

    idx_ref[...] = jnp.concatenate(idxs, axis=1) + b * N
    q_ref[0] = jnp.dot(tok.astype(jnp.bfloat16), qwT_ref[...].astype(jnp.bfloat16),
                       preferred_element_type=jnp.float32) + qb_ref[...]


def _kv_body(states_ref, kwT_ref, vwT_ref, kb_ref, vb_ref, kv_ref):
    st = states_ref[0]
    kv_ref[0, :, :SD] = jnp.dot(st, kwT_ref[...], preferred_element_type=jnp.float32) + kb_ref[...]
    kv_ref[0, :, SD:] = jnp.dot(st, vwT_ref[...], preferred_element_type=jnp.float32) + vb_ref[...]


def _sc_gather(table, idx):
    mesh = plsc.VectorSubcoreMesh(core_axis_name="c", subcore_axis_name="s")

    @functools.partial(
        pl.kernel,
        mesh=mesh,
        out_type=jax.ShapeDtypeStruct((M, KV), jnp.float32),
        scratch_types=[
            pltpu.VMEM((CH,), jnp.int32),
            pltpu.VMEM((CH, KV), jnp.float32),
            pltpu.SemaphoreType.DMA,
        ],
    )
    def gather_k(table_hbm, idx_hbm, out_hbm, idx_v, rows_v, sem):
        wid = lax.axis_index("s") * NC + lax.axis_index("c")
        base = wid * PER_W

        def body(i, carry):
            off = base + i * CH
            pltpu.sync_copy(idx_hbm.at[pl.ds(off, CH)], idx_v)
            pltpu.async_copy(table_hbm.at[idx_v], rows_v, sem).wait()
            pltpu.sync_copy(rows_v, out_hbm.at[pl.ds(off, CH)])
            return carry

        lax.fori_loop(0, PER_W // CH, body, 0)

    return gather_k(table, idx)


def _attn_body(q_ref, kv_ref, owT_ref, ob_ref, out_ref, attn_ref):
    q = q_ref[0]
    rows = kv_ref[...]

    d_iota = lax.broadcasted_iota(jnp.int32, (SD, H), 0)
    h_iota = lax.broadcasted_iota(jnp.int32, (SD, H), 1)
    dmat = (d_iota // HD == h_iota).astype(jnp.float32)
    h2 = lax.broadcasted_iota(jnp.int32, (H, SD), 0)
    d2 = lax.broadcasted_iota(jnp.int32, (H, SD), 1)
    dmat2 = (d2 // HD == h2).astype(jnp.float32)

    r_i = lax.broadcasted_iota(jnp.int32, (BSK, BS), 0)
    t_i = lax.broadcasted_iota(jnp.int32, (BSK, BS), 1)
    rep = (r_i // K == t_i).astype(jnp.bfloat16)
    t2 = lax.broadcasted_iota(jnp.int32, (BS, BSK), 0)
    r2 = lax.broadcasted_iota(jnp.int32, (BS, BSK), 1)
    seg = (r2 // K == t2).astype(jnp.float32)

    qrep = jnp.dot(rep, q.astype(jnp.bfloat16),
                   preferred_element_type=jnp.float32)
    prod = qrep * rows[:, :SD]
    logits = jnp.dot(prod, dmat, preferred_element_type=jnp.float32) * SCALE

    ex = jnp.exp(logits)
    denom = jnp.dot(seg, ex, preferred_element_type=jnp.float32)
    denom_rep = jnp.dot(rep.astype(jnp.float32), denom,
                        preferred_element_type=jnp.float32)
    w = ex / denom_rep
    attn_ref[0] = w

    wexp = jnp.dot(w, dmat2, preferred_element_type=jnp.float32)
    prod2 = (wexp * rows[:, SD:]).astype(jnp.bfloat16)
    o = jnp.dot(seg.astype(jnp.bfloat16), prod2,
                preferred_element_type=jnp.float32)

    out_ref[0] = jnp.dot(o.astype(jnp.bfloat16), owT_ref[...].astype(jnp.bfloat16),
                         preferred_element_type=jnp.float32) + ob_ref[...]


def kernel(tokens, states, router_w, router_b, q_w, q_b, k_w, k_b, v_w, v_b, out_w, out_b):
    statesT = states.transpose(0, 2, 1)
    rwT, qwT, kwT, vwT, owT = router_w.T, q_w.T, k_w.T, v_w.T, out_w.T
    rb = router_b.reshape(1, SD)
    qb = q_b.reshape(1, SD)
    kb = k_b.reshape(1, SD)
    vb = v_b.reshape(1, SD)
    ob = out_b.reshape(1, SD)

    idx, q = pl.pallas_call(
        _route_body,
        grid=(B, S // BS),
        in_specs=[
            pl.BlockSpec((1, BS, TD), lambda b, s: (b, s, 0)),
            pl.BlockSpec((1, SD, N), lambda b, s: (b, 0, 0)),
            pl.BlockSpec((TD, SD), lambda b, s: (0, 0)),
            pl.BlockSpec((1, SD), lambda b, s: (0, 0)),
            pl.BlockSpec((TD, SD), lambda b, s: (0, 0)),
            pl.BlockSpec((1, SD), lambda b, s: (0, 0)),
        ],
        out_specs=[
            pl.BlockSpec((BS, K), lambda b, s: (b * (S // BS) + s, 0)),
            pl.BlockSpec((1, BS, SD), lambda b, s: (b, s, 0)),
        ],
        out_shape=[
            jax.ShapeDtypeStruct((B * S, K), jnp.int32),
            jax.ShapeDtypeStruct((B, S, SD), jnp.float32),
        ],
    )(tokens, statesT, rwT, rb, qwT, qb)

    kv = pl.pallas_call(
        _kv_body,
        grid=(B,),
        in_specs=[
            pl.BlockSpec((1, N, SD), lambda b: (b, 0, 0)),
            pl.BlockSpec((SD, SD), lambda b: (0, 0)),
            pl.BlockSpec((SD, SD), lambda b: (0, 0)),
            pl.BlockSpec((1, SD), lambda b: (0, 0)),
            pl.BlockSpec((1, SD), lambda b: (0, 0)),
        ],
        out_specs=pl.BlockSpec((1, N, KV), lambda b: (b, 0, 0)),
        out_shape=jax.ShapeDtypeStruct((B, N, KV), jnp.float32),
    )(states, kwT, vwT, kb, vb)

    gathered = _sc_gather(kv.reshape(B * N, KV), idx.reshape(M))

    out, attn = pl.pallas_call(
        _attn_body,
        grid=(B, S // BS),
        in_specs=[
            pl.BlockSpec((1, BS, SD), lambda b, s: (b, s, 0)),
            pl.BlockSpec((BSK, KV), lambda b, s: (b * (S // BS) + s, 0)),
            pl.BlockSpec((SD, SD), lambda b, s: (0, 0)),
            pl.BlockSpec((1, SD), lambda b, s: (0, 0)),
        ],
        out_specs=[
            pl.BlockSpec((1, BS, SD), lambda b, s: (b, s, 0)),
            pl.BlockSpec((1, BSK, H), lambda b, s: (b, s, 0)),
        ],
        out_shape=[
            jax.ShapeDtypeStruct((B, S, SD), jnp.float32),
            jax.ShapeDtypeStruct((B, S * K, H), jnp.float32),
        ],
    )(q, gathered, owT, ob)

    attn_weights = attn.reshape(B, S, K, H).transpose(0, 3, 1, 2)
    return (out, attn_weights)


# speedup vs baseline: 84.7543x; 1.0590x over previous
import functools

import jax
import jax.numpy as jnp
from jax import lax
from jax.experimental import pallas as pl
from jax.experimental.pallas import tpu as pltpu
from jax.experimental.pallas import tpu_sc as plsc

B, S, N = 2, 2048, 1024
TD, SD, H, K = 1024, 1024, 16, 8
HD = SD // H
SCALE = HD ** (-0.5)
BS = 256
BSK = BS * K
M = B * S * K
KV = 2 * SD
NC, NS = 2, 16
NW = NC * NS
PER_W = M // NW
CH = 32


def _dotT(a, b):
    return lax.dot_general(a, b, (((1,), (1,)), ((), ())),
                           preferred_element_type=jnp.float32)


def _route_body(tok_ref, states_ref, rw_ref, rb_ref, qw_ref, qb_ref,
                idx_ref, q_ref):
    b = pl.program_id(0)
    tok = tok_ref[0]
    routed = _dotT(tok, rw_ref[...]) + rb_ref[...]
    scores = _dotT(routed, states_ref[0])

    iota_n = lax.broadcasted_iota(jnp.int32, (BS, N), 1)
    x = scores
    idxs = []
    for _ in range(K):
        m = jnp.max(x, axis=-1, keepdims=True)
        am = jnp.min(jnp.where(x == m, iota_n, N), axis=-1, keepdims=True)
        idxs.append(am)
        x = jnp.where(iota_n == am, -3e38, x)

    idx_ref[...] = jnp.concatenate(idxs, axis=1) + b * N
    q_ref[0] = _dotT(tok.astype(jnp.bfloat16),
                     qw_ref[...].astype(jnp.bfloat16)) + qb_ref[...]


def _kv_body(states_ref, kw_ref, vw_ref, kb_ref, vb_ref, kv_ref):
    st = states_ref[0]
    kv_ref[0, :, :SD] = _dotT(st, kw_ref[...]) + kb_ref[...]
    kv_ref[0, :, SD:] = _dotT(st, vw_ref[...]) + vb_ref[...]


def _sc_gather(table, idx):
    mesh = plsc.VectorSubcoreMesh(core_axis_name="c", subcore_axis_name="s")

    @functools.partial(
        pl.kernel,
        mesh=mesh,
        out_type=jax.ShapeDtypeStruct((M, KV), jnp.float32),
        scratch_types=[
            pltpu.VMEM((CH,), jnp.int32),
            pltpu.VMEM((CH, KV), jnp.float32),
            pltpu.SemaphoreType.DMA,
        ],
    )
    def gather_k(table_hbm, idx_hbm, out_hbm, idx_v, rows_v, sem):
        wid = lax.axis_index("s") * NC + lax.axis_index("c")
        base = wid * PER_W

        def body(i, carry):
            off = base + i * CH
            pltpu.sync_copy(idx_hbm.at[pl.ds(off, CH)], idx_v)
            pltpu.async_copy(table_hbm.at[idx_v], rows_v, sem).wait()
            pltpu.sync_copy(rows_v, out_hbm.at[pl.ds(off, CH)])
            return carry

        lax.fori_loop(0, PER_W // CH, body, 0)

    return gather_k(table, idx)


def _attn_body(q_ref, kv_ref, ow_ref, ob_ref, out_ref, attn_ref):
    q = q_ref[0]
    rows = kv_ref[...]

    d_iota = lax.broadcasted_iota(jnp.int32, (SD, H), 0)
    h_iota = lax.broadcasted_iota(jnp.int32, (SD, H), 1)
    dmat = (d_iota // HD == h_iota).astype(jnp.float32)
    h2 = lax.broadcasted_iota(jnp.int32, (H, SD), 0)
    d2 = lax.broadcasted_iota(jnp.int32, (H, SD), 1)
    dmat2 = (d2 // HD == h2).astype(jnp.float32)

    r_i = lax.broadcasted_iota(jnp.int32, (BSK, BS), 0)
    t_i = lax.broadcasted_iota(jnp.int32, (BSK, BS), 1)
    rep = (r_i // K == t_i).astype(jnp.bfloat16)
    t2 = lax.broadcasted_iota(jnp.int32, (BS, BSK), 0)
    r2 = lax.broadcasted_iota(jnp.int32, (BS, BSK), 1)
    seg = (r2 // K == t2).astype(jnp.float32)

    qrep = jnp.dot(rep, q.astype(jnp.bfloat16),
                   preferred_element_type=jnp.float32)
    prod = qrep * rows[:, :SD]
    logits = jnp.dot(prod, dmat, preferred_element_type=jnp.float32) * SCALE

    ex = jnp.exp(logits)
    denom = jnp.dot(seg, ex, preferred_element_type=jnp.float32)
    denom_rep = jnp.dot(rep.astype(jnp.float32), denom,
                        preferred_element_type=jnp.float32)
    w = ex / denom_rep
    attn_ref[0] = w

    wexp = jnp.dot(w, dmat2, preferred_element_type=jnp.float32)
    prod2 = (wexp * rows[:, SD:]).astype(jnp.bfloat16)
    o = jnp.dot(seg.astype(jnp.bfloat16), prod2,
                preferred_element_type=jnp.float32)

    out_ref[0] = _dotT(o.astype(jnp.bfloat16),
                       ow_ref[...].astype(jnp.bfloat16)) + ob_ref[...]


def kernel(tokens, states, router_w, router_b, q_w, q_b, k_w, k_b, v_w, v_b, out_w, out_b):
    rb = router_b.reshape(1, SD)
    qb = q_b.reshape(1, SD)
    kb = k_b.reshape(1, SD)
    vb = v_b.reshape(1, SD)
    ob = out_b.reshape(1, SD)

    idx, q = pl.pallas_call(
        _route_body,
        grid=(B, S // BS),
        in_specs=[
            pl.BlockSpec((1, BS, TD), lambda b, s: (b, s, 0)),
            pl.BlockSpec((1, N, SD), lambda b, s: (b, 0, 0)),
            pl.BlockSpec((TD, SD), lambda b, s: (0, 0)),
            pl.BlockSpec((1, SD), lambda b, s: (0, 0)),
            pl.BlockSpec((TD, SD), lambda b, s: (0, 0)),
            pl.BlockSpec((1, SD), lambda b, s: (0, 0)),
        ],
        out_specs=[
            pl.BlockSpec((BS, K), lambda b, s: (b * (S // BS) + s, 0)),
            pl.BlockSpec((1, BS, SD), lambda b, s: (b, s, 0)),
        ],
        out_shape=[
            jax.ShapeDtypeStruct((B * S, K), jnp.int32),
            jax.ShapeDtypeStruct((B, S, SD), jnp.float32),
        ],
    )(tokens, states, router_w, rb, q_w, qb)

    kv = pl.pallas_call(
        _kv_body,
        grid=(B,),
        in_specs=[
            pl.BlockSpec((1, N, SD), lambda b: (b, 0, 0)),
            pl.BlockSpec((SD, SD), lambda b: (0, 0)),
            pl.BlockSpec((SD, SD), lambda b: (0, 0)),
            pl.BlockSpec((1, SD), lambda b: (0, 0)),
            pl.BlockSpec((1, SD), lambda b: (0, 0)),
        ],
        out_specs=pl.BlockSpec((1, N, KV), lambda b: (b, 0, 0)),
        out_shape=jax.ShapeDtypeStruct((B, N, KV), jnp.float32),
    )(states, k_w, v_w, kb, vb)

    gathered = _sc_gather(kv.reshape(B * N, KV), idx.reshape(M))

    out, attn = pl.pallas_call(
        _attn_body,
        grid=(B, S // BS),
        in_specs=[
            pl.BlockSpec((1, BS, SD), lambda b, s: (b, s, 0)),
            pl.BlockSpec((BSK, KV), lambda b, s: (b * (S // BS) + s, 0)),
            pl.BlockSpec((SD, SD), lambda b, s: (0, 0)),
            pl.BlockSpec((1, SD), lambda b, s: (0, 0)),
        ],
        out_specs=[
            pl.BlockSpec((1, BS, SD), lambda b, s: (b, s, 0)),
            pl.BlockSpec((1, BSK, H), lambda b, s: (b, s, 0)),
        ],
        out_shape=[
            jax.ShapeDtypeStruct((B, S, SD), jnp.float32),
            jax.ShapeDtypeStruct((B, S * K, H), jnp.float32),
        ],
    )(q, gathered, out_w, ob)

    attn_weights = attn.reshape(B, S, K, H).transpose(0, 3, 1, 2)
    return (out, attn_weights)


# speedup vs baseline: 110.1717x; 1.2999x over previous
import functools

import jax
import jax.numpy as jnp
from jax import lax
from jax.experimental import pallas as pl
from jax.experimental.pallas import tpu as pltpu
from jax.experimental.pallas import tpu_sc as plsc

B, S, N = 2, 2048, 1024
TD, SD, H, K = 1024, 1024, 16, 8
HD = SD // H
SCALE = HD ** (-0.5)
BS = 256
BSK = BS * K
M = B * S * K
KV = 2 * SD
NC, NS = 2, 16
NW = NC * NS
PER_W = M // NW
CH = 64


def _dotT(a, b):
    return lax.dot_general(a, b, (((1,), (1,)), ((), ())),
                           preferred_element_type=jnp.float32)


def _route_body(tok_ref, states_ref, rw_ref, rb_ref, qw_ref, qb_ref,
                idx_ref, q_ref):
    b = pl.program_id(0)
    tok = tok_ref[0]
    routed = _dotT(tok, rw_ref[...]) + rb_ref[...]
    scores = _dotT(routed, states_ref[0])

    iota_n = lax.broadcasted_iota(jnp.int32, (BS, N), 1)
    x = scores
    idxs = []
    for _ in range(K):
        m = jnp.max(x, axis=-1, keepdims=True)
        am = jnp.min(jnp.where(x == m, iota_n, N), axis=-1, keepdims=True)
        idxs.append(am)
        x = jnp.where(iota_n == am, -3e38, x)

    idx_ref[...] = jnp.concatenate(idxs, axis=1) + b * N
    q_ref[0] = _dotT(tok.astype(jnp.bfloat16),
                     qw_ref[...].astype(jnp.bfloat16)) + qb_ref[...]


def _pack(x):
    xb = x.astype(jnp.bfloat16)
    lo = lax.bitcast_convert_type(xb[:, :SD // 2], jnp.uint16).astype(jnp.uint32)
    hi = lax.bitcast_convert_type(xb[:, SD // 2:], jnp.uint16).astype(jnp.uint32)
    return lax.bitcast_convert_type(lo | (hi << 16), jnp.int32)


def _unpack(x):
    xu = lax.bitcast_convert_type(x, jnp.uint32)
    lo = lax.bitcast_convert_type((xu & 0xFFFF).astype(jnp.uint16), jnp.bfloat16)
    hi = lax.bitcast_convert_type((xu >> 16).astype(jnp.uint16), jnp.bfloat16)
    return jnp.concatenate([lo, hi], axis=1)


def _kv_body(states_ref, kw_ref, vw_ref, kb_ref, vb_ref, kv_ref):
    st = states_ref[0]
    kv_ref[0, :, :SD // 2] = _pack(_dotT(st, kw_ref[...]) + kb_ref[...])
    kv_ref[0, :, SD // 2:] = _pack(_dotT(st, vw_ref[...]) + vb_ref[...])


def _sc_gather(table, idx):
    mesh = plsc.VectorSubcoreMesh(core_axis_name="c", subcore_axis_name="s")

    @functools.partial(
        pl.kernel,
        mesh=mesh,
        out_type=jax.ShapeDtypeStruct((M, SD), jnp.int32),
        scratch_types=[
            pltpu.VMEM((CH,), jnp.int32),
            pltpu.VMEM((CH, SD), jnp.int32),
            pltpu.SemaphoreType.DMA,
        ],
    )
    def gather_k(table_hbm, idx_hbm, out_hbm, idx_v, rows_v, sem):
        wid = lax.axis_index("s") * NC + lax.axis_index("c")
        base = wid * PER_W

        def body(i, carry):
            off = base + i * CH
            pltpu.sync_copy(idx_hbm.at[pl.ds(off, CH)], idx_v)
            pltpu.async_copy(table_hbm.at[idx_v], rows_v, sem).wait()
            pltpu.sync_copy(rows_v, out_hbm.at[pl.ds(off, CH)])
            return carry

        lax.fori_loop(0, PER_W // CH, body, 0)

    return gather_k(table, idx)


def _attn_body(q_ref, kv_ref, ow_ref, ob_ref, out_ref, attn_ref):
    q = q_ref[0]
    packed = kv_ref[...]
    krows = _unpack(packed[:, :SD // 2]).astype(jnp.float32)
    vrows = _unpack(packed[:, SD // 2:]).astype(jnp.float32)

    d_iota = lax.broadcasted_iota(jnp.int32, (SD, H), 0)
    h_iota = lax.broadcasted_iota(jnp.int32, (SD, H), 1)
    dmat = (d_iota // HD == h_iota).astype(jnp.float32)
    h2 = lax.broadcasted_iota(jnp.int32, (H, SD), 0)
    d2 = lax.broadcasted_iota(jnp.int32, (H, SD), 1)
    dmat2 = (d2 // HD == h2).astype(jnp.float32)

    r_i = lax.broadcasted_iota(jnp.int32, (BSK, BS), 0)
    t_i = lax.broadcasted_iota(jnp.int32, (BSK, BS), 1)
    rep = (r_i // K == t_i).astype(jnp.bfloat16)
    t2 = lax.broadcasted_iota(jnp.int32, (BS, BSK), 0)
    r2 = lax.broadcasted_iota(jnp.int32, (BS, BSK), 1)
    seg = (r2 // K == t2).astype(jnp.float32)

    qrep = jnp.dot(rep, q.astype(jnp.bfloat16),
                   preferred_element_type=jnp.float32)
    prod = qrep * krows
    logits = jnp.dot(prod, dmat, preferred_element_type=jnp.float32) * SCALE

    ex = jnp.exp(logits)
    denom = jnp.dot(seg, ex, preferred_element_type=jnp.float32)
    denom_rep = jnp.dot(rep.astype(jnp.float32), denom,
                        preferred_element_type=jnp.float32)
    w = ex / denom_rep
    attn_ref[0] = w

    wexp = jnp.dot(w, dmat2, preferred_element_type=jnp.float32)
    prod2 = (wexp * vrows).astype(jnp.bfloat16)
    o = jnp.dot(seg.astype(jnp.bfloat16), prod2,
                preferred_element_type=jnp.float32)

    out_ref[0] = _dotT(o.astype(jnp.bfloat16),
                       ow_ref[...].astype(jnp.bfloat16)) + ob_ref[...]


def kernel(tokens, states, router_w, router_b, q_w, q_b, k_w, k_b, v_w, v_b, out_w, out_b):
    rb = router_b.reshape(1, SD)
    qb = q_b.reshape(1, SD)
    kb = k_b.reshape(1, SD)
    vb = v_b.reshape(1, SD)
    ob = out_b.reshape(1, SD)

    idx, q = pl.pallas_call(
        _route_body,
        grid=(B, S // BS),
        in_specs=[
            pl.BlockSpec((1, BS, TD), lambda b, s: (b, s, 0)),
            pl.BlockSpec((1, N, SD), lambda b, s: (b, 0, 0)),
            pl.BlockSpec((TD, SD), lambda b, s: (0, 0)),
            pl.BlockSpec((1, SD), lambda b, s: (0, 0)),
            pl.BlockSpec((TD, SD), lambda b, s: (0, 0)),
            pl.BlockSpec((1, SD), lambda b, s: (0, 0)),
        ],
        out_specs=[
            pl.BlockSpec((BS, K), lambda b, s: (b * (S // BS) + s, 0)),
            pl.BlockSpec((1, BS, SD), lambda b, s: (b, s, 0)),
        ],
        out_shape=[
            jax.ShapeDtypeStruct((B * S, K), jnp.int32),
            jax.ShapeDtypeStruct((B, S, SD), jnp.float32),
        ],
    )(tokens, states, router_w, rb, q_w, qb)

    kv = pl.pallas_call(
        _kv_body,
        grid=(B,),
        in_specs=[
            pl.BlockSpec((1, N, SD), lambda b: (b, 0, 0)),
            pl.BlockSpec((SD, SD), lambda b: (0, 0)),
            pl.BlockSpec((SD, SD), lambda b: (0, 0)),
            pl.BlockSpec((1, SD), lambda b: (0, 0)),
            pl.BlockSpec((1, SD), lambda b: (0, 0)),
        ],
        out_specs=pl.BlockSpec((1, N, SD), lambda b: (b, 0, 0)),
        out_shape=jax.ShapeDtypeStruct((B, N, SD), jnp.int32),
    )(states, k_w, v_w, kb, vb)

    gathered = _sc_gather(kv.reshape(B * N, SD), idx.reshape(M))

    out, attn = pl.pallas_call(
        _attn_body,
        grid=(B, S // BS),
        in_specs=[
            pl.BlockSpec((1, BS, SD), lambda b, s: (b, s, 0)),
            pl.BlockSpec((BSK, SD), lambda b, s: (b * (S // BS) + s, 0)),
            pl.BlockSpec((SD, SD), lambda b, s: (0, 0)),
            pl.BlockSpec((1, SD), lambda b, s: (0, 0)),
        ],
        out_specs=[
            pl.BlockSpec((1, BS, SD), lambda b, s: (b, s, 0)),
            pl.BlockSpec((1, BSK, H), lambda b, s: (b, s, 0)),
        ],
        out_shape=[
            jax.ShapeDtypeStruct((B, S, SD), jnp.float32),
            jax.ShapeDtypeStruct((B, S * K, H), jnp.float32),
        ],
    )(q, gathered, out_w, ob)

    attn_weights = attn.reshape(B, S, K, H).transpose(0, 3, 1, 2)
    return (out, attn_weights)


# speedup vs baseline: 117.5291x; 1.0668x over previous
import functools

import jax
import jax.numpy as jnp
from jax import lax
from jax.experimental import pallas as pl
from jax.experimental.pallas import tpu as pltpu
from jax.experimental.pallas import tpu_sc as plsc

B, S, N = 2, 2048, 1024
TD, SD, H, K = 1024, 1024, 16, 8
HD = SD // H
SCALE = HD ** (-0.5)
BS = 256
BSK = BS * K
MB = S * K
NC, NS = 2, 16
NW = NC * NS
PER_W = MB // NW
CH = 64


def _dotT(a, b):
    return lax.dot_general(a, b, (((1,), (1,)), ((), ())),
                           preferred_element_type=jnp.float32)


def _route_body(tok_ref, states_ref, rw_ref, rb_ref, qw_ref, qb_ref,
                idx_ref, q_ref):
    tok = tok_ref[...]
    routed = _dotT(tok, rw_ref[...]) + rb_ref[...]
    scores = _dotT(routed, states_ref[...])

    iota_n = lax.broadcasted_iota(jnp.int32, (BS, N), 1)
    x = scores
    idxs = []
    for _ in range(K):
        m = jnp.max(x, axis=-1, keepdims=True)
        am = jnp.min(jnp.where(x == m, iota_n, N), axis=-1, keepdims=True)
        idxs.append(am)
        x = jnp.where(iota_n == am, -3e38, x)

    idx_ref[...] = jnp.concatenate(idxs, axis=1)
    q_ref[...] = _dotT(tok.astype(jnp.bfloat16),
                       qw_ref[...].astype(jnp.bfloat16)) + qb_ref[...]


def _pack(x):
    xb = x.astype(jnp.bfloat16)
    lo = lax.bitcast_convert_type(xb[:, :SD // 2], jnp.uint16).astype(jnp.uint32)
    hi = lax.bitcast_convert_type(xb[:, SD // 2:], jnp.uint16).astype(jnp.uint32)
    return lax.bitcast_convert_type(lo | (hi << 16), jnp.int32)


def _unpack(x):
    xu = lax.bitcast_convert_type(x, jnp.uint32)
    lo = lax.bitcast_convert_type((xu & 0xFFFF).astype(jnp.uint16), jnp.bfloat16)
    hi = lax.bitcast_convert_type((xu >> 16).astype(jnp.uint16), jnp.bfloat16)
    return jnp.concatenate([lo, hi], axis=1)


def _kv_body(states_ref, kw_ref, vw_ref, kb_ref, vb_ref, kv_ref):
    st = states_ref[0]
    kv_ref[0, :, :SD // 2] = _pack(_dotT(st, kw_ref[...]) + kb_ref[...])
    kv_ref[0, :, SD // 2:] = _pack(_dotT(st, vw_ref[...]) + vb_ref[...])


def _sc_gather(table, idx):
    mesh = plsc.VectorSubcoreMesh(core_axis_name="c", subcore_axis_name="s")

    @functools.partial(
        pl.kernel,
        mesh=mesh,
        out_type=jax.ShapeDtypeStruct((MB, SD), jnp.int32),
        scratch_types=[
            pltpu.VMEM((CH,), jnp.int32),
            pltpu.VMEM((CH, SD), jnp.int32),
            pltpu.SemaphoreType.DMA,
        ],
    )
    def gather_k(table_hbm, idx_hbm, out_hbm, idx_v, rows_v, sem):
        wid = lax.axis_index("s") * NC + lax.axis_index("c")
        base = wid * PER_W

        def body(i, carry):
            off = base + i * CH
            pltpu.sync_copy(idx_hbm.at[pl.ds(off, CH)], idx_v)
            pltpu.async_copy(table_hbm.at[idx_v], rows_v, sem).wait()
            pltpu.sync_copy(rows_v, out_hbm.at[pl.ds(off, CH)])
            return carry

        lax.fori_loop(0, PER_W // CH, body, 0)

    return gather_k(table, idx)


def _attn_body(q_ref, kv_ref, ow_ref, ob_ref, out_ref, attn_ref):
    q = q_ref[...]
    packed = kv_ref[...]
    krows = _unpack(packed[:, :SD // 2]).astype(jnp.float32)
    vrows = _unpack(packed[:, SD // 2:]).astype(jnp.float32)

    d_iota = lax.broadcasted_iota(jnp.int32, (SD, H), 0)
    h_iota = lax.broadcasted_iota(jnp.int32, (SD, H), 1)
    dmat = (d_iota // HD == h_iota).astype(jnp.float32)
    h2 = lax.broadcasted_iota(jnp.int32, (H, SD), 0)
    d2 = lax.broadcasted_iota(jnp.int32, (H, SD), 1)
    dmat2 = (d2 // HD == h2).astype(jnp.float32)

    r_i = lax.broadcasted_iota(jnp.int32, (BSK, BS), 0)
    t_i = lax.broadcasted_iota(jnp.int32, (BSK, BS), 1)
    rep = (r_i // K == t_i).astype(jnp.bfloat16)
    t2 = lax.broadcasted_iota(jnp.int32, (BS, BSK), 0)
    r2 = lax.broadcasted_iota(jnp.int32, (BS, BSK), 1)
    seg = (r2 // K == t2).astype(jnp.float32)

    qrep = jnp.dot(rep, q.astype(jnp.bfloat16),
                   preferred_element_type=jnp.float32)
    prod = qrep * krows
    logits = jnp.dot(prod, dmat, preferred_element_type=jnp.float32) * SCALE

    ex = jnp.exp(logits)
    denom = jnp.dot(seg, ex, preferred_element_type=jnp.float32)
    denom_rep = jnp.dot(rep.astype(jnp.float32), denom,
                        preferred_element_type=jnp.float32)
    w = ex / denom_rep
    attn_ref[...] = w

    wexp = jnp.dot(w, dmat2, preferred_element_type=jnp.float32)
    prod2 = (wexp * vrows).astype(jnp.bfloat16)
    o = jnp.dot(seg.astype(jnp.bfloat16), prod2,
                preferred_element_type=jnp.float32)

    out_ref[...] = _dotT(o.astype(jnp.bfloat16),
                         ow_ref[...].astype(jnp.bfloat16)) + ob_ref[...]


def kernel(tokens, states, router_w, router_b, q_w, q_b, k_w, k_b, v_w, v_b, out_w, out_b):
    rb = router_b.reshape(1, SD)
    qb = q_b.reshape(1, SD)
    kb = k_b.reshape(1, SD)
    vb = v_b.reshape(1, SD)
    ob = out_b.reshape(1, SD)

    kv = pl.pallas_call(
        _kv_body,
        grid=(B,),
        in_specs=[
            pl.BlockSpec((1, N, SD), lambda b: (b, 0, 0)),
            pl.BlockSpec((SD, SD), lambda b: (0, 0)),
            pl.BlockSpec((SD, SD), lambda b: (0, 0)),
            pl.BlockSpec((1, SD), lambda b: (0, 0)),
            pl.BlockSpec((1, SD), lambda b: (0, 0)),
        ],
        out_specs=pl.BlockSpec((1, N, SD), lambda b: (b, 0, 0)),
        out_shape=jax.ShapeDtypeStruct((B, N, SD), jnp.int32),
    )(states, k_w, v_w, kb, vb)

    route = pl.pallas_call(
        _route_body,
        grid=(S // BS,),
        in_specs=[
            pl.BlockSpec((BS, TD), lambda s: (s, 0)),
            pl.BlockSpec((N, SD), lambda s: (0, 0)),
            pl.BlockSpec((TD, SD), lambda s: (0, 0)),
            pl.BlockSpec((1, SD), lambda s: (0, 0)),
            pl.BlockSpec((TD, SD), lambda s: (0, 0)),
            pl.BlockSpec((1, SD), lambda s: (0, 0)),
        ],
        out_specs=[
            pl.BlockSpec((BS, K), lambda s: (s, 0)),
            pl.BlockSpec((BS, SD), lambda s: (s, 0)),
        ],
        out_shape=[
            jax.ShapeDtypeStruct((S, K), jnp.int32),
            jax.ShapeDtypeStruct((S, SD), jnp.float32),
        ],
    )

    attn = pl.pallas_call(
        _attn_body,
        grid=(S // BS,),
        in_specs=[
            pl.BlockSpec((BS, SD), lambda s: (s, 0)),
            pl.BlockSpec((BSK, SD), lambda s: (s, 0)),
            pl.BlockSpec((SD, SD), lambda s: (0, 0)),
            pl.BlockSpec((1, SD), lambda s: (0, 0)),
        ],
        out_specs=[
            pl.BlockSpec((BS, SD), lambda s: (s, 0)),
            pl.BlockSpec((BSK, H), lambda s: (s, 0)),
        ],
        out_shape=[
            jax.ShapeDtypeStruct((S, SD), jnp.float32),
            jax.ShapeDtypeStruct((MB, H), jnp.float32),
        ],
    )

    outs, attns = [], []
    for b in range(B):
        idx_b, q_b2 = route(tokens[b], states[b], router_w, rb, q_w, qb)
        gathered = _sc_gather(kv[b], idx_b.reshape(MB))
        out_b2, attn_b = attn(q_b2, gathered, out_w, ob)
        outs.append(out_b2[None])
        attns.append(attn_b[None])

    out = jnp.concatenate(outs, axis=0)
    attn_w = jnp.concatenate(attns, axis=0).reshape(B, S, K, H)
    return (out, attn_w.transpose(0, 3, 1, 2))


# speedup vs baseline: 118.7018x; 1.0100x over previous
import functools

import jax
import jax.numpy as jnp
from jax import lax
from jax.experimental import pallas as pl
from jax.experimental.pallas import tpu as pltpu
from jax.experimental.pallas import tpu_sc as plsc

B, S, N = 2, 2048, 1024
TD, SD, H, K = 1024, 1024, 16, 8
HD = SD // H
SCALE = HD ** (-0.5)
BS = 256
RBS = 512
BSK = BS * K
MB = S * K
NC, NS = 2, 16
NW = NC * NS
PER_W = MB // NW
CH = 64


def _dotT(a, b):
    return lax.dot_general(a, b, (((1,), (1,)), ((), ())),
                           preferred_element_type=jnp.float32)


def _route_body(tok_ref, states_ref, rw_ref, rb_ref, qw_ref, qb_ref,
                idx_ref, q_ref):
    tok = tok_ref[...]
    routed = _dotT(tok, rw_ref[...]) + rb_ref[...]
    scores = _dotT(routed, states_ref[...])

    iota_n = lax.broadcasted_iota(jnp.int32, (RBS, N), 1)
    x = scores
    idxs = []
    for _ in range(K):
        m = jnp.max(x, axis=-1, keepdims=True)
        am = jnp.min(jnp.where(x == m, iota_n, N), axis=-1, keepdims=True)
        idxs.append(am)
        x = jnp.where(iota_n == am, -3e38, x)

    idx_ref[...] = jnp.concatenate(idxs, axis=1)
    q_ref[...] = _dotT(tok.astype(jnp.bfloat16),
                       qw_ref[...].astype(jnp.bfloat16)) + qb_ref[...]


def _pack(x):
    xb = x.astype(jnp.bfloat16)
    lo = lax.bitcast_convert_type(xb[:, :SD // 2], jnp.uint16).astype(jnp.uint32)
    hi = lax.bitcast_convert_type(xb[:, SD // 2:], jnp.uint16).astype(jnp.uint32)
    return lax.bitcast_convert_type(lo | (hi << 16), jnp.int32)


def _unpack(x):
    xu = lax.bitcast_convert_type(x, jnp.uint32)
    lo = lax.bitcast_convert_type((xu & 0xFFFF).astype(jnp.uint16), jnp.bfloat16)
    hi = lax.bitcast_convert_type((xu >> 16).astype(jnp.uint16), jnp.bfloat16)
    return jnp.concatenate([lo, hi], axis=1)


def _kv_body(states_ref, kw_ref, vw_ref, kb_ref, vb_ref, kv_ref):
    st = states_ref[0]
    kv_ref[0, :, :SD // 2] = _pack(_dotT(st, kw_ref[...]) + kb_ref[...])
    kv_ref[0, :, SD // 2:] = _pack(_dotT(st, vw_ref[...]) + vb_ref[...])


def _sc_gather(table, idx):
    mesh = plsc.VectorSubcoreMesh(core_axis_name="c", subcore_axis_name="s")

    @functools.partial(
        pl.kernel,
        mesh=mesh,
        out_type=jax.ShapeDtypeStruct((MB, SD), jnp.int32),
        scratch_types=[
            pltpu.VMEM((CH,), jnp.int32),
            pltpu.VMEM((CH, SD), jnp.int32),
            pltpu.SemaphoreType.DMA,
        ],
    )
    def gather_k(table_hbm, idx_hbm, out_hbm, idx_v, rows_v, sem):
        wid = lax.axis_index("s") * NC + lax.axis_index("c")
        base = wid * PER_W

        def body(i, carry):
            off = base + i * CH
            pltpu.sync_copy(idx_hbm.at[pl.ds(off, CH)], idx_v)
            pltpu.async_copy(table_hbm.at[idx_v], rows_v, sem).wait()
            pltpu.sync_copy(rows_v, out_hbm.at[pl.ds(off, CH)])
            return carry

        lax.fori_loop(0, PER_W // CH, body, 0)

    return gather_k(table, idx)


def _attn_body(q_ref, kv_ref, ow_ref, ob_ref, out_ref, attn_ref):
    q = q_ref[...]
    packed = kv_ref[...]
    krows = _unpack(packed[:, :SD // 2]).astype(jnp.float32)
    vrows = _unpack(packed[:, SD // 2:]).astype(jnp.float32)

    d_iota = lax.broadcasted_iota(jnp.int32, (SD, H), 0)
    h_iota = lax.broadcasted_iota(jnp.int32, (SD, H), 1)
    dmat = (d_iota // HD == h_iota).astype(jnp.float32)
    h2 = lax.broadcasted_iota(jnp.int32, (H, SD), 0)
    d2 = lax.broadcasted_iota(jnp.int32, (H, SD), 1)
    dmat2 = (d2 // HD == h2).astype(jnp.float32)

    r_i = lax.broadcasted_iota(jnp.int32, (BSK, BS), 0)
    t_i = lax.broadcasted_iota(jnp.int32, (BSK, BS), 1)
    rep = (r_i // K == t_i).astype(jnp.bfloat16)
    t2 = lax.broadcasted_iota(jnp.int32, (BS, BSK), 0)
    r2 = lax.broadcasted_iota(jnp.int32, (BS, BSK), 1)
    seg = (r2 // K == t2).astype(jnp.float32)

    qrep = jnp.dot(rep, q.astype(jnp.bfloat16),
                   preferred_element_type=jnp.float32)
    prod = qrep * krows
    logits = jnp.dot(prod, dmat, preferred_element_type=jnp.float32) * SCALE

    ex = jnp.exp(logits)
    denom = jnp.dot(seg, ex, preferred_element_type=jnp.float32)
    denom_rep = jnp.dot(rep.astype(jnp.float32), denom,
                        preferred_element_type=jnp.float32)
    w = ex / denom_rep
    attn_ref[...] = w

    wexp = jnp.dot(w, dmat2, preferred_element_type=jnp.float32)
    prod2 = (wexp * vrows).astype(jnp.bfloat16)
    o = jnp.dot(seg.astype(jnp.bfloat16), prod2,
                preferred_element_type=jnp.float32)

    out_ref[...] = _dotT(o.astype(jnp.bfloat16),
                         ow_ref[...].astype(jnp.bfloat16)) + ob_ref[...]


def kernel(tokens, states, router_w, router_b, q_w, q_b, k_w, k_b, v_w, v_b, out_w, out_b):
    rb = router_b.reshape(1, SD)
    qb = q_b.reshape(1, SD)
    kb = k_b.reshape(1, SD)
    vb = v_b.reshape(1, SD)
    ob = out_b.reshape(1, SD)

    kv = pl.pallas_call(
        _kv_body,
        grid=(B,),
        in_specs=[
            pl.BlockSpec((1, N, SD), lambda b: (b, 0, 0)),
            pl.BlockSpec((SD, SD), lambda b: (0, 0)),
            pl.BlockSpec((SD, SD), lambda b: (0, 0)),
            pl.BlockSpec((1, SD), lambda b: (0, 0)),
            pl.BlockSpec((1, SD), lambda b: (0, 0)),
        ],
        out_specs=pl.BlockSpec((1, N, SD), lambda b: (b, 0, 0)),
        out_shape=jax.ShapeDtypeStruct((B, N, SD), jnp.int32),
    )(states, k_w, v_w, kb, vb)

    route = pl.pallas_call(
        _route_body,
        grid=(S // RBS,),
        in_specs=[
            pl.BlockSpec((RBS, TD), lambda s: (s, 0)),
            pl.BlockSpec((N, SD), lambda s: (0, 0)),
            pl.BlockSpec((TD, SD), lambda s: (0, 0)),
            pl.BlockSpec((1, SD), lambda s: (0, 0)),
            pl.BlockSpec((TD, SD), lambda s: (0, 0)),
            pl.BlockSpec((1, SD), lambda s: (0, 0)),
        ],
        out_specs=[
            pl.BlockSpec((RBS, K), lambda s: (s, 0)),
            pl.BlockSpec((RBS, SD), lambda s: (s, 0)),
        ],
        out_shape=[
            jax.ShapeDtypeStruct((S, K), jnp.int32),
            jax.ShapeDtypeStruct((S, SD), jnp.float32),
        ],
    )

    attn = pl.pallas_call(
        _attn_body,
        grid=(S // BS,),
        in_specs=[
            pl.BlockSpec((BS, SD), lambda s: (s, 0)),
            pl.BlockSpec((BSK, SD), lambda s: (s, 0)),
            pl.BlockSpec((SD, SD), lambda s: (0, 0)),
            pl.BlockSpec((1, SD), lambda s: (0, 0)),
        ],
        out_specs=[
            pl.BlockSpec((BS, SD), lambda s: (s, 0)),
            pl.BlockSpec((BSK, H), lambda s: (s, 0)),
        ],
        out_shape=[
            jax.ShapeDtypeStruct((S, SD), jnp.float32),
            jax.ShapeDtypeStruct((MB, H), jnp.float32),
        ],
    )

    gathered, qs = [], []
    for b in range(B):
        idx_b, q_b2 = route(tokens[b], states[b], router_w, rb, q_w, qb)
        qs.append(q_b2)
        gathered.append(_sc_gather(kv[b], idx_b.reshape(MB)))

    outs, attns = [], []
    for b in range(B):
        out_b2, attn_b = attn(qs[b], gathered[b], out_w, ob)
        outs.append(out_b2[None])
        attns.append(attn_b[None])

    out = jnp.concatenate(outs, axis=0)
    attn_w = jnp.concatenate(attns, axis=0).reshape(B, S, K, H)
    return (out, attn_w.transpose(0, 3, 1, 2))
